# Initial kernel scaffold; baseline (speedup 1.0000x reference)
import jax
import jax.numpy as jnp
from jax.experimental import pallas as pl


def kernel(user_x, item_x, user_factor_0, item_factor_0, edge_u_q_u, edge_i_q_i, edge_u_i, edge_i_u, params):
    raise NotImplementedError("write your pallas kernel here")


# speedup vs baseline: 1.0290x; 1.0290x over previous
import functools

import jax
import jax.numpy as jnp
from jax.experimental import pallas as pl
from jax.experimental.pallas import tpu as pltpu

NUM_USERS = 20000
NUM_ITEMS = 20000
HIDDEN = 16
HEADS = 4
OUT_D = HIDDEN * HEADS


def _project(x, p):
    return x @ p["W"].T + p["b"]


def _conv_accum_xla(h, asrc, adst, edges, lo, hi):
    src, dst = edges[0], edges[1]
    e = jax.nn.leaky_relu(asrc[src] + adst[dst], negative_slope=0.2)
    p = jnp.exp(e)
    valid = (dst >= lo) & (dst < hi)
    seg = jnp.where(valid, dst - lo, hi - lo)
    n_out = hi - lo
    p = jnp.where(valid[:, None], p, 0.0)
    msg = h[src].reshape(-1, HEADS, HIDDEN) * p[:, :, None]
    denom = jax.ops.segment_sum(p, seg, num_segments=n_out + 1)[:n_out]
    num = jax.ops.segment_sum(msg, seg, num_segments=n_out + 1)[:n_out]
    return num, denom


def _epilogue_kernel(num_ref, den_ref, self_h_ref, self_p_ref, bias_ref, out_ref):
    num = num_ref[...]
    den = den_ref[...]
    sh = self_h_ref[...]
    sp = self_p_ref[...]
    n = num + sh * sp[:, :, None]
    d = den + sp
    out = n / (d[:, :, None] + 1e-16)
    out_ref[...] = out.reshape(out.shape[0], OUT_D) + bias_ref[...]


def _finish(num, denom, h, asrc, adst, bias, lo, hi):
    n_out = hi - lo
    h_self = h[lo:hi].reshape(n_out, HEADS, HIDDEN)
    p_self = jnp.exp(jax.nn.leaky_relu(asrc[lo:hi] + adst[lo:hi], negative_slope=0.2))
    B = 2000
    grid = (n_out // B,)
    return pl.pallas_call(
        _epilogue_kernel,
        grid=grid,
        in_specs=[
            pl.BlockSpec((B, HEADS, HIDDEN), lambda i: (i, 0, 0)),
            pl.BlockSpec((B, HEADS), lambda i: (i, 0)),
            pl.BlockSpec((B, HEADS, HIDDEN), lambda i: (i, 0, 0)),
            pl.BlockSpec((B, HEADS), lambda i: (i, 0)),
            pl.BlockSpec((OUT_D,), lambda i: (0,)),
        ],
        out_specs=pl.BlockSpec((B, OUT_D), lambda i: (i, 0)),
        out_shape=jax.ShapeDtypeStruct((n_out, OUT_D), jnp.float32),
    )(num, denom, h_self, p_self, bias)


def _gat(x, edges, p, lo, hi):
    h = x @ p["W"].T
    hh = h.reshape(-1, HEADS, HIDDEN)
    asrc = jnp.sum(hh * p["a_src"], axis=-1)
    adst = jnp.sum(hh * p["a_dst"], axis=-1)
    num, denom = _conv_accum_xla(h, asrc, adst, edges, lo, hi)
    return _finish(num, denom, h, asrc, adst, p["b"], lo, hi)


def kernel(user_x, item_x, user_factor_0, item_factor_0, edge_u_q_u, edge_i_q_i, edge_u_i, edge_i_u, params):
    user_emb = _project(user_x, params["user_proj"])
    item_emb = _project(item_x, params["item_proj"])
    uf = _project(user_factor_0, params["user_factor_proj"])
    itf = _project(item_factor_0, params["item_factor_proj"])
    all_emb = jnp.concatenate([user_emb, item_emb, uf, itf], axis=0)

    H_u = _gat(all_emb, edge_u_q_u, params["gat_u_q_u"], 0, NUM_USERS)
    H_i = _gat(all_emb, edge_i_q_i, params["gat_i_q_i"], NUM_USERS, NUM_USERS + NUM_ITEMS)
    combined = jnp.concatenate([H_u, H_i], axis=0)

    H_hat_u = _gat(combined, edge_u_i, params["gat_i_u"], 0, NUM_USERS)
    H_hat_i = _gat(combined, edge_i_u, params["gat_u_i"], NUM_USERS, NUM_USERS + NUM_ITEMS)
    return H_hat_u, H_hat_i


# speedup vs baseline: 40.4048x; 39.2651x over previous
import functools

import jax
import jax.numpy as jnp
from jax import lax
from jax.experimental import pallas as pl
from jax.experimental.pallas import tpu as pltpu
from jax.experimental.pallas import tpu_sc as plsc

NUM_USERS = 20000
NUM_ITEMS = 20000
HIDDEN = 16
HEADS = 4
OUT_D = HIDDEN * HEADS
SROW = 80
DROW = 16
RANGE = 20000
R_ACC = 20096
DUMP = 20000
CHUNK = 128
NW = 32


def _proj_body(x_ref, w_ref, b_ref, o_ref):
    o_ref[...] = x_ref[...] @ w_ref[...].T + b_ref[...]


def _proj(x, p):
    n, _ = x.shape
    blk = 2000 if n % 2000 == 0 else 1000
    return pl.pallas_call(
        _proj_body,
        grid=(n // blk,),
        in_specs=[
            pl.BlockSpec((blk, x.shape[1]), lambda i: (i, 0)),
            pl.BlockSpec(p["W"].shape, lambda i: (0, 0)),
            pl.BlockSpec((1, HIDDEN), lambda i: (0, 0)),
        ],
        out_specs=pl.BlockSpec((blk, HIDDEN), lambda i: (i, 0)),
        out_shape=jax.ShapeDtypeStruct((n, HIDDEN), jnp.float32),
    )(x, p["W"], p["b"].reshape(1, HIDDEN))


def _feat_body(x_ref, w_ref, aa_ref, h_ref, ab_ref):
    h = x_ref[...] @ w_ref[...].T
    h_ref[...] = h
    ab_ref[...] = h @ aa_ref[...]


def _feat(x, p, blk=2000):
    n, in_d = x.shape
    eye = jnp.eye(HEADS, dtype=jnp.float32)
    a_src = (p["a_src"][0].T[None, :, :] * eye[:, None, :]).reshape(OUT_D, HEADS)
    a_dst = (p["a_dst"][0].T[None, :, :] * eye[:, None, :]).reshape(OUT_D, HEADS)
    aa = jnp.concatenate([a_src, a_dst], axis=1)
    return pl.pallas_call(
        _feat_body,
        grid=(n // blk,),
        in_specs=[
            pl.BlockSpec((blk, in_d), lambda i: (i, 0)),
            pl.BlockSpec((OUT_D, in_d), lambda i: (0, 0)),
            pl.BlockSpec((OUT_D, 2 * HEADS), lambda i: (0, 0)),
        ],
        out_specs=[
            pl.BlockSpec((blk, OUT_D), lambda i: (i, 0)),
            pl.BlockSpec((blk, 2 * HEADS), lambda i: (i, 0)),
        ],
        out_shape=[
            jax.ShapeDtypeStruct((n, OUT_D), jnp.float32),
            jax.ShapeDtypeStruct((n, 2 * HEADS), jnp.float32),
        ],
    )(x, p["W"], aa)


def _epi_body(a0_ref, a1_ref, h_ref, ab_ref, b_ref, o_ref):
    acc = a0_ref[0] + a1_ref[0]
    num = acc[:, :OUT_D]
    den = acc[:, OUT_D:OUT_D + HEADS]
    ab = ab_ref[...]
    a = ab[:, :HEADS] + ab[:, HEADS:]
    ps = jnp.exp(jnp.where(a >= 0, a, 0.2 * a))
    hm = h_ref[...]
    psb = jnp.repeat(ps, HIDDEN, axis=1)
    num = num + hm * psb
    den = den + ps
    denb = jnp.repeat(den + 1e-16, HIDDEN, axis=1)
    o_ref[...] = num / denb + b_ref[...]


def _epilogue(acc, h_self, ab_self, bias, blk=2000):
    return pl.pallas_call(
        _epi_body,
        grid=(RANGE // blk,),
        in_specs=[
            pl.BlockSpec((1, blk, SROW), lambda i: (0, i, 0)),
            pl.BlockSpec((1, blk, SROW), lambda i: (1, i, 0)),
            pl.BlockSpec((blk, OUT_D), lambda i: (i, 0)),
            pl.BlockSpec((blk, 2 * HEADS), lambda i: (i, 0)),
            pl.BlockSpec((1, OUT_D), lambda i: (0, 0)),
        ],
        out_specs=pl.BlockSpec((blk, OUT_D), lambda i: (i, 0)),
        out_shape=jax.ShapeDtypeStruct((RANGE, OUT_D), jnp.float32),
    )(acc, acc, h_self, ab_self, bias.reshape(1, OUT_D))


def _epi4(a_ref, b_ref, o_ref):
    o_ref[...] = a_ref[...] + b_ref[...]


@functools.lru_cache(maxsize=None)
def _make_sc_conv(e_pad, lo):
    per_tile = e_pad // NW
    n_chunks = per_tile // CHUNK
    rows_per_tile = R_ACC // 16
    mesh = plsc.VectorSubcoreMesh(core_axis_name="c", subcore_axis_name="s")

    @functools.partial(
        pl.kernel,
        out_type=jax.ShapeDtypeStruct((2, R_ACC, SROW), jnp.float32),
        mesh=mesh,
        compiler_params=pltpu.CompilerParams(use_tc_tiling_on_sc=False),
        scratch_types=[
            pltpu.VMEM((CHUNK,), jnp.int32),
            pltpu.VMEM((CHUNK,), jnp.int32),
            pltpu.VMEM((CHUNK,), jnp.int32),
            pltpu.VMEM((CHUNK,), jnp.int32),
            pltpu.VMEM((CHUNK, SROW), jnp.float32),
            pltpu.VMEM((CHUNK, DROW), jnp.float32),
            pltpu.VMEM_SHARED((R_ACC, SROW), jnp.float32),
            pltpu.SemaphoreType.DMA,
            pltpu.SemaphoreType.DMA,
        ],
    )
    def sc_conv(src_hbm, dst_hbm, s_hbm, d_hbm, zero_hbm, out_hbm,
                srcv, dstv, idxl, dstc, sbuf, dbuf, accum, sem1, sem2):
        c = lax.axis_index("c")
        s = lax.axis_index("s")
        wid = s * 2 + c
        r0 = s * rows_per_tile
        pltpu.sync_copy(zero_hbm.at[pl.ds(r0, rows_per_tile)],
                        accum.at[pl.ds(r0, rows_per_tile)])
        plsc.subcore_barrier()
        base_w = wid * per_tile

        def chunk_body(g, carry):
            base = base_w + g * CHUNK
            pltpu.sync_copy(src_hbm.at[pl.ds(base, CHUNK)], srcv)
            pltpu.sync_copy(dst_hbm.at[pl.ds(base, CHUNK)], dstv)
            for v in range(CHUNK // 16):
                d = dstv[pl.ds(16 * v, 16)]
                valid = (d >= lo) & (d < lo + RANGE)
                idxl[pl.ds(16 * v, 16)] = jnp.where(valid, d - lo, DUMP)
                dstc[pl.ds(16 * v, 16)] = jnp.where(valid, d, 0)
            pltpu.async_copy(s_hbm.at[srcv], sbuf, sem1).wait()
            pltpu.async_copy(d_hbm.at[dstc], dbuf, sem2).wait()

            def edge_body(i, carry2):
                a = sbuf[i, pl.ds(OUT_D, 16)] + dbuf[i, pl.ds(0, 16)]
                e = jnp.where(a >= 0, a, 0.2 * a)
                p = jnp.exp(e)
                sbuf[i, pl.ds(OUT_D, 16)] = p
                for k in range(HEADS):
                    pk = p[k]
                    sbuf[i, pl.ds(16 * k, 16)] = sbuf[i, pl.ds(16 * k, 16)] * pk
                return carry2

            lax.fori_loop(0, CHUNK, edge_body, 0)
            pltpu.sync_copy(sbuf, accum.at[idxl], add=True)
            return carry

        lax.fori_loop(0, n_chunks, chunk_body, 0)
        plsc.subcore_barrier()
        pltpu.sync_copy(accum.at[pl.ds(r0, rows_per_tile)],
                        out_hbm.at[c, pl.ds(r0, rows_per_tile)])

    return sc_conv


def _gat(h, ab, edges, bias, lo, zero_acc):
    n = h.shape[0]
    e = edges.shape[1]
    e_pad = ((e + NW * CHUNK - 1) // (NW * CHUNK)) * (NW * CHUNK)
    pad = e_pad - e
    src = jnp.concatenate([edges[0], jnp.zeros((pad,), jnp.int32)])
    dst = jnp.concatenate([edges[1], jnp.full((pad,), -1, jnp.int32)])
    s_tab = jnp.concatenate(
        [h, ab[:, :HEADS], jnp.zeros((n, SROW - OUT_D - HEADS), jnp.float32)], axis=1)
    d_tab = jnp.concatenate(
        [ab[:, HEADS:], jnp.zeros((n, DROW - HEADS), jnp.float32)], axis=1)
    acc = _make_sc_conv(e_pad, lo)(src, dst, s_tab, d_tab, zero_acc)
    sl = slice(lo, lo + RANGE)
    return _epilogue(acc, h[sl], ab[sl], bias)


def kernel(user_x, item_x, user_factor_0, item_factor_0,
           edge_u_q_u, edge_i_q_i, edge_u_i, edge_i_u, params):
    zero_acc = jnp.zeros((R_ACC, SROW), jnp.float32)
    all_emb = jnp.concatenate([
        _proj(user_x, params["user_proj"]),
        _proj(item_x, params["item_proj"]),
        _proj(user_factor_0, params["user_factor_proj"]),
        _proj(item_factor_0, params["item_factor_proj"]),
    ], axis=0)

    h1, ab1 = _feat(all_emb, params["gat_u_q_u"])
    h2, ab2 = _feat(all_emb, params["gat_i_q_i"])
    H_u = _gat(h1, ab1, edge_u_q_u, params["gat_u_q_u"]["b"], 0, zero_acc)
    H_i = _gat(h2, ab2, edge_i_q_i, params["gat_i_q_i"]["b"], NUM_USERS, zero_acc)
    combined = jnp.concatenate([H_u, H_i], axis=0)

    h3, ab3 = _feat(combined, params["gat_i_u"])
    h4, ab4 = _feat(combined, params["gat_u_i"])
    H_hat_u = _gat(h3, ab3, edge_u_i, params["gat_i_u"]["b"], 0, zero_acc)
    H_hat_i = _gat(h4, ab4, edge_i_u, params["gat_u_i"]["b"], NUM_USERS, zero_acc)
    return H_hat_u, H_hat_i
